# speedup vs baseline: 1.4679x; 1.4679x over previous
import functools

import jax
import jax.numpy as jnp
from jax import lax
from jax.experimental import pallas as pl
from jax.experimental.pallas import tpu as pltpu


def _features_kernel(xt_ref, w1, b1, w2, b2, w3, b3, w4, b4, w5, b5,
                     pooled_ref, *, n):
    r = xt_ref.shape[1]
    h = lax.dot_general(
        xt_ref[...], w1[...],
        dimension_numbers=(((0,), (0,)), ((), ())),
        preferred_element_type=jnp.float32)
    h = jnp.maximum(h + b1[...], 0.0).astype(jnp.bfloat16)

    def fused(h, w_ref, b_ref):
        z = jnp.dot(h, w_ref[...],
                    preferred_element_type=jnp.float32) + b_ref[...]
        return jnp.maximum(z, 0.0)

    h = fused(h, w2, b2).astype(jnp.bfloat16)
    h = fused(h, w3, b3).astype(jnp.bfloat16)
    h = fused(h, w4, b4).astype(jnp.bfloat16)
    h = fused(h, w5, b5)

    pooled_ref[...] = jnp.max(h.reshape(r // n, n, h.shape[-1]), axis=1)


def _head_kernel(p_ref, w6, b6, w7, b7, out_ref):
    h = jnp.dot(p_ref[...], w6[...],
                preferred_element_type=jnp.float32) + b6[...]
    h = jnp.maximum(h, 0.0)
    out_ref[...] = (jnp.dot(h, w7[...],
                            preferred_element_type=jnp.float32) + b7[...])


def kernel(x_ncw, w_0, b_0, w_1, b_1, w_2, b_2, w_3, b_3, w_4, b_4,
           w_5, b_5, w7, b7):
    B, cin, N = x_ncw.shape
    emb = w_4.shape[1]
    out_c = w7.shape[1]

    bblk = 32
    grid_b = -(-B // bblk)
    Bp = grid_b * bblk
    if Bp != B:
        x_ncw = jnp.pad(x_ncw, ((0, Bp - B), (0, 0), (0, 0)))

    xt = jnp.transpose(x_ncw, (1, 0, 2)).reshape(cin, Bp * N)
    xt = xt.astype(jnp.bfloat16)
    R = bblk * N

    conv_ws = [w.astype(jnp.bfloat16) for w in (w_0, w_1, w_2, w_3, w_4)]
    conv_bs = [b_0, b_1, b_2, b_3, b_4]

    in_specs = [pl.BlockSpec((cin, R), lambda i: (0, i))]
    args = [xt]
    for w, b in zip(conv_ws, conv_bs):
        args += [w, b]
        in_specs += [pl.BlockSpec(w.shape, lambda i: (0, 0)),
                     pl.BlockSpec(b.shape, lambda i: (0, 0))]

    pooled = pl.pallas_call(
        functools.partial(_features_kernel, n=N),
        out_shape=jax.ShapeDtypeStruct((Bp, emb), jnp.float32),
        grid=(grid_b,),
        in_specs=in_specs,
        out_specs=pl.BlockSpec((bblk, emb), lambda i: (i, 0)),
        compiler_params=pltpu.CompilerParams(
            dimension_semantics=("parallel",)),
    )(*args)

    out_pad = ((out_c + 127) // 128) * 128
    w7p = jnp.pad(w7, ((0, 0), (0, out_pad - out_c)))
    b7p = jnp.pad(b7, ((0, 0), (0, out_pad - out_c)))

    hblk = Bp // 2
    logits = pl.pallas_call(
        _head_kernel,
        out_shape=jax.ShapeDtypeStruct((Bp, out_pad), jnp.float32),
        grid=(2,),
        in_specs=[pl.BlockSpec((hblk, emb), lambda i: (i, 0)),
                  pl.BlockSpec(w_5.shape, lambda i: (0, 0)),
                  pl.BlockSpec(b_5.shape, lambda i: (0, 0)),
                  pl.BlockSpec(w7p.shape, lambda i: (0, 0)),
                  pl.BlockSpec(b7p.shape, lambda i: (0, 0))],
        out_specs=pl.BlockSpec((hblk, out_pad), lambda i: (i, 0)),
        compiler_params=pltpu.CompilerParams(
            dimension_semantics=("parallel",)),
    )(pooled, w_5, b_5, w7p, b7p)

    return logits[:B, :out_c]


# speedup vs baseline: 1.8623x; 1.2687x over previous
import functools

import jax
import jax.numpy as jnp
from jax import lax
from jax.experimental import pallas as pl
from jax.experimental.pallas import tpu as pltpu


def _features_kernel(xp_ref, w1, b1, w2, b2, w3, b3, w4, b4, w5, b5,
                     pooled_ref, *, n, half, emb):
    rh = xp_ref.shape[1]
    zero = jnp.bfloat16(0.0)

    z = lax.dot_general(
        xp_ref[...], w1[...],
        dimension_numbers=(((0,), (0,)), ((), ())),
        preferred_element_type=jnp.float32)
    h = jnp.maximum(z.astype(jnp.bfloat16) + b1[...], zero)

    def fused(h, w_ref, b_ref):
        z = jnp.dot(h, w_ref[...], preferred_element_type=jnp.float32)
        return jnp.maximum(z.astype(jnp.bfloat16) + b_ref[...], zero)

    h = fused(h, w2, b2)
    h = fused(h, w3, b3)
    h = fused(h, w4, b4)
    z5 = jnp.dot(h, w5[...], preferred_element_type=jnp.float32)

    pooled = jnp.max(z5.reshape(half, n, 2 * emb), axis=1)
    pooled = jnp.maximum(pooled + b5[...], 0.0)
    pooled_ref[...] = jnp.concatenate(
        [pooled[:, :emb], pooled[:, emb:]], axis=0)


def _head_kernel(p_ref, w6, b6, w7, b7, out_ref):
    h = jnp.dot(p_ref[...], w6[...],
                preferred_element_type=jnp.float32) + b6[...]
    h = jnp.maximum(h, 0.0)
    out_ref[...] = (jnp.dot(h, w7[...],
                            preferred_element_type=jnp.float32) + b7[...])


def _bdiag(w):
    z = jnp.zeros(w.shape, w.dtype)
    return jnp.concatenate(
        [jnp.concatenate([w, z], axis=1),
         jnp.concatenate([z, w], axis=1)], axis=0)


def kernel(x_ncw, w_0, b_0, w_1, b_1, w_2, b_2, w_3, b_3, w_4, b_4,
           w_5, b_5, w7, b7):
    B, cin, N = x_ncw.shape
    emb = w_4.shape[1]
    out_c = w7.shape[1]

    bblk = 32
    half = bblk // 2
    grid_b = -(-B // bblk)
    Bp = grid_b * bblk
    if Bp != B:
        x_ncw = jnp.pad(x_ncw, ((0, Bp - B), (0, 0), (0, 0)))

    rh = half * N
    xp = (x_ncw.transpose(1, 0, 2)
          .reshape(cin, grid_b, 2, rh)
          .transpose(2, 0, 1, 3)
          .reshape(2 * cin, grid_b * rh)
          .astype(jnp.bfloat16))

    w1p = jnp.zeros((2 * cin, 128), jnp.float32)
    w1p = w1p.at[:cin, :64].set(w_0).at[cin:, 64:].set(w_0)
    conv_ws = [w1p.astype(jnp.bfloat16)] + [
        _bdiag(w).astype(jnp.bfloat16) for w in (w_1, w_2, w_3, w_4)]
    dup = lambda b: jnp.concatenate([b, b], axis=1)
    conv_bs = [dup(b).astype(jnp.bfloat16) for b in (b_0, b_1, b_2, b_3)]
    conv_bs.append(dup(b_4))

    in_specs = [pl.BlockSpec((2 * cin, rh), lambda i: (0, i))]
    args = [xp]
    for w, b in zip(conv_ws, conv_bs):
        args += [w, b]
        in_specs += [pl.BlockSpec(w.shape, lambda i: (0, 0)),
                     pl.BlockSpec(b.shape, lambda i: (0, 0))]

    pooled = pl.pallas_call(
        functools.partial(_features_kernel, n=N, half=half, emb=emb),
        out_shape=jax.ShapeDtypeStruct((Bp, emb), jnp.float32),
        grid=(grid_b,),
        in_specs=in_specs,
        out_specs=pl.BlockSpec((bblk, emb), lambda i: (i, 0)),
        compiler_params=pltpu.CompilerParams(
            dimension_semantics=("parallel",)),
    )(*args)

    out_pad = ((out_c + 127) // 128) * 128
    w7p = jnp.pad(w7, ((0, 0), (0, out_pad - out_c)))
    b7p = jnp.pad(b7, ((0, 0), (0, out_pad - out_c)))

    hblk = Bp // 2
    logits = pl.pallas_call(
        _head_kernel,
        out_shape=jax.ShapeDtypeStruct((Bp, out_pad), jnp.float32),
        grid=(2,),
        in_specs=[pl.BlockSpec((hblk, emb), lambda i: (i, 0)),
                  pl.BlockSpec(w_5.shape, lambda i: (0, 0)),
                  pl.BlockSpec(b_5.shape, lambda i: (0, 0)),
                  pl.BlockSpec(w7p.shape, lambda i: (0, 0)),
                  pl.BlockSpec(b7p.shape, lambda i: (0, 0))],
        out_specs=pl.BlockSpec((hblk, out_pad), lambda i: (i, 0)),
        compiler_params=pltpu.CompilerParams(
            dimension_semantics=("parallel",)),
    )(pooled, w_5, b_5, w7p, b7p)

    return logits[:B, :out_c]
